# baseline (speedup 1.0000x reference)
import jax
import jax.numpy as jnp
from jax import lax
from jax.experimental import pallas as pl
from jax.experimental.pallas import tpu as pltpu
from jax.experimental.pallas import tpu_sc as plsc

VOCAB = 1000000
B = 16384
L = 20
D = 64
NC = 2
NS = 16
NW = NC * NS
BPW = B // NW
SUB = 32
NSUB = BPW // SUB
CPS = SUB * L
BG = 4
IDXW = 128


def _perm(v, idx):
    return jnp.take_along_axis(v, idx, axis=0, mode="promise_in_bounds")


def _tree_reduce16(accs, perms, masks, brev):
    vs = accs
    for s, d in enumerate((8, 4, 2, 1)):
        m, p = masks[s], perms[s]
        vs = [jnp.where(m, vs[2 * i], vs[2 * i + 1])
              + _perm(jnp.where(m, vs[2 * i + 1], vs[2 * i]), p)
              for i in range(len(vs) // 2)]
    return _perm(vs[0], brev)


def _sc_body(wp_r, wb_r, cp_r, cb_r, we2, ce2, out_hbm,
             idx_w, wb_v, idx_c, cb_v, w_sel, c_rows, out_c, sem):
    wid = lax.axis_index("c") * NS + lax.axis_index("s")

    lane = lax.iota(jnp.int32, 16)
    perms = []
    masks = []
    for d in (8, 4, 2, 1):
        perms.append((lane & ~(2 * d - 1)) | ((lane + d) & (2 * d - 1)))
        masks.append((lane % (2 * d)) < d)
    brev = (((lane & 1) << 3) | ((lane & 2) << 1)
            | (((lane & 4) >> 1) | ((lane & 8) >> 3)))

    pltpu.sync_copy(wp_r.at[pl.ds(wid * 4, 4)], idx_w)
    pltpu.sync_copy(wb_r.at[pl.ds(wid * 4, 4)], wb_v)

    for t in range(4):
        pltpu.async_copy(we2.at[idx_w.at[t]],
                         c_rows.at[pl.ds(0, IDXW)], sem).wait()

        def wsel_body(r, _, t=t):
            bits = wb_v[t, pl.ds((r // 16) * 16, 16)]
            m = _perm(bits, jnp.full((16,), r % 16, jnp.int32))
            for k in range(4):
                lo = c_rows[r, pl.ds(k * 16, 16)]
                hi = c_rows[r, pl.ds(D + k * 16, 16)]
                w_sel[t * IDXW + r, pl.ds(k * 16, 16)] = lo + (hi - lo) * m
            return ()

        lax.fori_loop(0, IDXW, wsel_body, (), unroll=False)

    def chunk_body(sub, _):
        pltpu.sync_copy(cp_r.at[pl.ds(wid * 80 + sub * 5, 5)], idx_c)
        pltpu.sync_copy(cb_r.at[pl.ds(wid * 80 + sub * 5, 5)], cb_v)
        c_copies = []
        for j in range(5):
            c_copies.append(pltpu.async_copy(
                ce2.at[idx_c.at[j]],
                c_rows.at[pl.ds(j * IDXW, IDXW)], sem))
        for c in c_copies:
            c.wait()

        def group_body(bg, _):
            row0 = sub * SUB + bg * BG
            wv = [[w_sel[row0 + bi, pl.ds(k * 16, 16)] for k in range(4)]
                  for bi in range(BG)]
            cbase = bg * (BG * L)
            for g in range(5):
                accs_lo = []
                accs_hi = []
                for o in range(16):
                    f = g * 16 + o
                    cr = cbase + f
                    bi = f // L
                    alo = wv[bi][0] * c_rows[cr, pl.ds(0, 16)]
                    ahi = wv[bi][0] * c_rows[cr, pl.ds(D, 16)]
                    for k in range(1, 4):
                        alo = alo + wv[bi][k] * c_rows[cr, pl.ds(k * 16, 16)]
                        ahi = ahi + wv[bi][k] * c_rows[cr, pl.ds(D + k * 16, 16)]
                    accs_lo.append(alo)
                    accs_hi.append(ahi)
                res_lo = _tree_reduce16(accs_lo, perms, masks, brev)
                res_hi = _tree_reduce16(accs_hi, perms, masks, brev)
                fl = cbase + g * 16
                m1 = cb_v[fl // IDXW, pl.ds(fl % IDXW, 16)]
                out_c[pl.ds(fl, 16)] = res_lo + (res_hi - res_lo) * m1
            return ()

        lax.fori_loop(0, SUB // BG, group_body, (), unroll=False)

        pltpu.sync_copy(out_c,
                        out_hbm.at[pl.ds(wid * BPW * L + sub * CPS, CPS)])
        return ()

    lax.fori_loop(0, NSUB, chunk_body, (), unroll=False)


@jax.jit
def _word2vec_sc(wp_r, wb_r, cp_r, cb_r, we2, ce2):
    mesh = plsc.VectorSubcoreMesh(core_axis_name="c", subcore_axis_name="s")
    return pl.kernel(
        _sc_body,
        out_type=jax.ShapeDtypeStruct((B * L,), jnp.float32),
        mesh=mesh,
        compiler_params=pltpu.CompilerParams(use_tc_tiling_on_sc=False),
        scratch_types=[
            pltpu.VMEM((4, IDXW), jnp.int32),
            pltpu.VMEM((4, IDXW), jnp.float32),
            pltpu.VMEM((5, IDXW), jnp.int32),
            pltpu.VMEM((5, IDXW), jnp.float32),
            pltpu.VMEM((BPW, D), jnp.float32),
            pltpu.VMEM((CPS, 2 * D), jnp.float32),
            pltpu.VMEM((CPS,), jnp.float32),
            pltpu.SemaphoreType.DMA,
        ],
    )(wp_r, wb_r, cp_r, cb_r, we2, ce2)


def kernel(word_ids, context_ids, word_embed, context_embed):
    we2 = word_embed.reshape(VOCAB // 2, 2 * D)
    ce2 = context_embed.reshape(VOCAB // 2, 2 * D)
    wp_r = (word_ids >> 1).reshape(B // IDXW, IDXW)
    wb_r = (word_ids & 1).astype(jnp.float32).reshape(B // IDXW, IDXW)
    cflat = context_ids.reshape(B * L)
    cp_r = (cflat >> 1).reshape(B * L // IDXW, IDXW)
    cb_r = (cflat & 1).astype(jnp.float32).reshape(B * L // IDXW, IDXW)
    return _word2vec_sc(wp_r, wb_r, cp_r, cb_r, we2, ce2).reshape(B, L)
